# baseline (speedup 1.0000x reference)
import functools

import jax
import jax.numpy as jnp
from jax import lax
from jax.experimental import pallas as pl
from jax.experimental.pallas import tpu as pltpu
from jax.experimental.pallas import tpu_sc as plsc

_INV_T = 2.0
_K = 64
_NC, _NS, _L = 2, 16, 16
_NW = _NC * _NS
_CH = 1024


def _sq_body(x_hbm, tab_hbm, assign_hbm, qst_hbm,
             tab_v, pp_v, qq_v, rr_v, ss_v,
             x_v, ui_v, vi_v, q_v, out_v):
    n = x_hbm.shape[0]
    per_w = n // _NW
    nchunk = per_w // _CH
    wid = lax.axis_index("s") * _NC + lax.axis_index("c")

    pltpu.sync_copy(tab_hbm, tab_v)
    pltpu.sync_copy(tab_hbm.at[pl.ds(2 * _K, _K)], pp_v)
    pltpu.sync_copy(tab_hbm.at[pl.ds(3 * _K, _K)], qq_v)
    pltpu.sync_copy(tab_hbm.at[pl.ds(4 * _K, _K)], rr_v)
    pltpu.sync_copy(tab_hbm.at[pl.ds(5 * _K, _K)], ss_v)

    a_blk = [tab_v[pl.ds(b * _L, _L)] for b in range(_K // _L)]
    b_blk = [tab_v[pl.ds(_K + b * _L, _L)] for b in range(_K // _L)]
    scal = tab_v[pl.ds(6 * _K, _L)]
    c0 = scal[0]
    inv_d = scal[1]
    cmax = scal[2]

    base0 = wid * per_w

    def chunk_body(ci, carry):
        base = base0 + ci * _CH
        pltpu.sync_copy(x_hbm.at[pl.ds(base, _CH)], x_v)

        def phase1(i, c1):
            xv = x_v[pl.ds(i * _L, _L)]
            xc = jnp.minimum(jnp.maximum(xv, c0), cmax)
            ii = jnp.minimum(((xc - c0) * inv_d).astype(jnp.int32), _K - 1)
            u = jnp.exp(-_INV_T * xc)
            v = jnp.exp(_INV_T * xc)
            pg = plsc.load_gather(pp_v, [ii])
            qg = plsc.load_gather(qq_v, [ii])
            rg = plsc.load_gather(rr_v, [ii])
            sg = plsc.load_gather(ss_v, [ii])
            inv = 1.0 / (u * pg + v * qg)
            quant = (u * rg + v * sg) * inv
            q_v[pl.ds(i * _L, _L)] = xv + (quant - xv)
            ui_v[pl.ds(i * _L, _L)] = u * inv
            vi_v[pl.ds(i * _L, _L)] = v * inv
            return c1

        lax.fori_loop(0, _CH // _L, phase1, 0, unroll=2)

        def phase2(i, c2):
            uvec = ui_v[pl.ds(i * _L, _L)]
            vvec = vi_v[pl.ds(i * _L, _L)]
            for j in range(_L):
                ab = jnp.full((_L,), uvec[j], jnp.float32)
                bb = jnp.full((_L,), vvec[j], jnp.float32)
                row = i * _L + j
                for blk in range(_K // _L):
                    e = jnp.minimum(ab * a_blk[blk], bb * b_blk[blk])
                    out_v[pl.ds(row * _K + blk * _L, _L)] = e
            return c2

        lax.fori_loop(0, _CH // _L, phase2, 0)

        pltpu.sync_copy(out_v, assign_hbm.at[pl.ds(base * _K, _CH * _K)])
        pltpu.sync_copy(q_v, qst_hbm.at[pl.ds(base, _CH)])
        return carry

    lax.fori_loop(0, nchunk, chunk_body, 0)


def kernel(x, centers):
    shape = x.shape
    n = x.size
    assert n % (_NW * _CH) == 0, n

    c = centers.astype(jnp.float32)
    a_t = jnp.exp(_INV_T * c)
    b_t = jnp.exp(-_INV_T * c)
    p_t = jnp.cumsum(a_t)
    q_t = jnp.cumsum(b_t[::-1])[::-1] - b_t
    r_t = jnp.cumsum(a_t * c)
    s_t = jnp.cumsum((b_t * c)[::-1])[::-1] - b_t * c
    inv_d = (_K - 1) / (c[-1] - c[0])
    scal = jnp.zeros((_K,), jnp.float32)
    scal = scal.at[0].set(c[0]).at[1].set(inv_d).at[2].set(c[-1])
    tab = jnp.concatenate([a_t, b_t, p_t, q_t, r_t, s_t, scal,
                           jnp.zeros((_K,), jnp.float32)])

    run = pl.kernel(
        _sq_body,
        out_type=[
            jax.ShapeDtypeStruct((n * _K,), jnp.float32),
            jax.ShapeDtypeStruct((n,), jnp.float32),
        ],
        mesh=plsc.VectorSubcoreMesh(core_axis_name="c", subcore_axis_name="s"),
        compiler_params=pltpu.CompilerParams(needs_layout_passes=False),
        scratch_types=[
            pltpu.VMEM((8 * _K,), jnp.float32),
            pltpu.VMEM((_K,), jnp.float32),
            pltpu.VMEM((_K,), jnp.float32),
            pltpu.VMEM((_K,), jnp.float32),
            pltpu.VMEM((_K,), jnp.float32),
            pltpu.VMEM((_CH,), jnp.float32),
            pltpu.VMEM((_CH,), jnp.float32),
            pltpu.VMEM((_CH,), jnp.float32),
            pltpu.VMEM((_CH,), jnp.float32),
            pltpu.VMEM((_CH * _K,), jnp.float32),
        ],
    )
    assign, qst = run(x.reshape(-1), tab)
    return qst.reshape(shape), assign.reshape(shape + (_K,))


# speedup vs baseline: 3.3373x; 3.3373x over previous
import jax
import jax.numpy as jnp
from jax import lax
from jax.experimental import pallas as pl
from jax.experimental.pallas import tpu as pltpu
from jax.experimental.pallas import tpu_sc as plsc

_INV_T = 2.0
_K = 64
_NC, _NS, _L = 2, 16, 16
_NW = _NC * _NS
_C = 384
_CHI = _C // 128
_PIX_W = _K * _CHI * 128
_GRP_W = 8 * _CHI * 128


def _sq_body(x_hbm, tab_hbm, assign_hbm, qst_hbm,
             tab_v, pp_v, qq_v, rr_v, ss_v,
             xg_v, qg_v, out0_v, out1_v, sem0, sem1):
    npix = qst_hbm.shape[0] // _C
    grp_per_w = npix // (8 * _NW)
    wid = lax.axis_index("s") * _NC + lax.axis_index("c")

    pltpu.sync_copy(tab_hbm, tab_v)
    pltpu.sync_copy(tab_hbm.at[pl.ds(0, _K)], pp_v)
    pltpu.sync_copy(tab_hbm.at[pl.ds(_K, _K)], qq_v)
    pltpu.sync_copy(tab_hbm.at[pl.ds(2 * _K, _K)], rr_v)
    pltpu.sync_copy(tab_hbm.at[pl.ds(3 * _K, _K)], ss_v)

    scal = tab_v[pl.ds(4 * _K, _L)]
    c0 = scal[0]
    inv_d = scal[1]
    cmax = scal[2]
    a0s = jnp.full((_L,), scal[3], jnp.float32)
    b0s = jnp.full((_L,), scal[4], jnp.float32)
    gs = jnp.full((_L,), scal[5], jnp.float32)
    gis = jnp.full((_L,), scal[6], jnp.float32)

    out_bufs = (out0_v, out1_v)
    sems = (sem0, sem1)

    def group_body(g, carry):
        grp = wid * grp_per_w + g
        pltpu.sync_copy(x_hbm.at[pl.ds(grp * _GRP_W, _GRP_W)], xg_v)

        def pair_body(pair, c2):
            for b in range(2):
                wl = pair * 2 + b
                out_v = out_bufs[b]
                sem = sems[b]

                @pl.when(jnp.logical_or(g > 0, pair > 0))
                def _():
                    pltpu.make_async_copy(
                        assign_hbm.at[pl.ds(0, _PIX_W)], out_v, sem).wait()

                for chi in range(_CHI):
                    def cv_body(j, c3, chi=chi, wl=wl, out_v=out_v):
                        xoff = chi * 1024 + wl * 128 + j * _L
                        xv = xg_v[pl.ds(xoff, _L)]
                        xc = jnp.minimum(jnp.maximum(xv, c0), cmax)
                        ii = jnp.minimum(
                            ((xc - c0) * inv_d).astype(jnp.int32), _K - 1)
                        u = jnp.exp(-_INV_T * xc)
                        v = jnp.exp(_INV_T * xc)
                        pg = plsc.load_gather(pp_v, [ii])
                        qg = plsc.load_gather(qq_v, [ii])
                        rg = plsc.load_gather(rr_v, [ii])
                        sg = plsc.load_gather(ss_v, [ii])
                        inv = 1.0 / (u * pg + v * qg)
                        quant = (u * rg + v * sg) * inv
                        qg_v[pl.ds(xoff, _L)] = xv + (quant - xv)
                        ua = (u * inv) * a0s
                        vb = (v * inv) * b0s
                        obase = chi * 1024 + j * _L
                        for kh in range(8):
                            for kl in range(8):
                                e = jnp.minimum(ua, vb)
                                out_v[pl.ds(obase + kh * 3072 + kl * 128,
                                            _L)] = e
                                ua = ua * gs
                                vb = vb * gis
                        return c3

                    lax.fori_loop(0, 128 // _L, cv_body, 0)

                pltpu.async_copy(
                    out_v,
                    assign_hbm.at[pl.ds((grp * 8 + wl) * _PIX_W, _PIX_W)],
                    sem)
            return c2

        lax.fori_loop(0, 4, pair_body, 0)
        pltpu.sync_copy(qg_v, qst_hbm.at[pl.ds(grp * _GRP_W, _GRP_W)])
        return carry

    lax.fori_loop(0, grp_per_w, group_body, 0)

    pltpu.make_async_copy(assign_hbm.at[pl.ds(0, _PIX_W)], out0_v, sem0).wait()
    pltpu.make_async_copy(assign_hbm.at[pl.ds(0, _PIX_W)], out1_v, sem1).wait()


def kernel(x, centers):
    b, c, h, w = x.shape
    assert c == _C and (b * h * w) % (8 * _NW) == 0, x.shape
    n = x.size

    cf = centers.astype(jnp.float32)
    a_t = jnp.exp(_INV_T * cf)
    b_t = jnp.exp(-_INV_T * cf)
    p_t = jnp.cumsum(a_t)
    q_t = jnp.cumsum(b_t[::-1])[::-1] - b_t
    r_t = jnp.cumsum(a_t * cf)
    s_t = jnp.cumsum((b_t * cf)[::-1])[::-1] - b_t * cf
    inv_d = (_K - 1) / (cf[-1] - cf[0])
    step = (cf[-1] - cf[0]) / (_K - 1)
    scal = jnp.zeros((_K,), jnp.float32)
    scal = (scal.at[0].set(cf[0]).at[1].set(inv_d).at[2].set(cf[-1])
            .at[3].set(a_t[0]).at[4].set(b_t[0])
            .at[5].set(jnp.exp(_INV_T * step)).at[6].set(jnp.exp(-_INV_T * step)))
    tab = jnp.concatenate([p_t, q_t, r_t, s_t, scal])

    x1d = (x.transpose(0, 2, 3, 1)
            .reshape(b, h, w // 8, 8, _CHI, 128)
            .transpose(0, 1, 2, 4, 3, 5)
            .reshape(-1))

    run = pl.kernel(
        _sq_body,
        out_type=[
            jax.ShapeDtypeStruct((n * _K,), jnp.float32),
            jax.ShapeDtypeStruct((n,), jnp.float32),
        ],
        mesh=plsc.VectorSubcoreMesh(core_axis_name="c", subcore_axis_name="s"),
        compiler_params=pltpu.CompilerParams(needs_layout_passes=False),
        scratch_types=[
            pltpu.VMEM((5 * _K,), jnp.float32),
            pltpu.VMEM((_K,), jnp.float32),
            pltpu.VMEM((_K,), jnp.float32),
            pltpu.VMEM((_K,), jnp.float32),
            pltpu.VMEM((_K,), jnp.float32),
            pltpu.VMEM((_GRP_W,), jnp.float32),
            pltpu.VMEM((_GRP_W,), jnp.float32),
            pltpu.VMEM((_PIX_W,), jnp.float32),
            pltpu.VMEM((_PIX_W,), jnp.float32),
            pltpu.SemaphoreType.DMA,
            pltpu.SemaphoreType.DMA,
        ],
    )
    assign1d, q1d = run(x1d, tab)

    assign = (assign1d.reshape(b, h, w, 8, _CHI, 8, 128)
              .transpose(0, 4, 6, 1, 2, 3, 5)
              .reshape(b, c, h, w, _K))
    qst = (q1d.reshape(b, h, w // 8, _CHI, 8, 128)
           .transpose(0, 3, 5, 1, 2, 4)
           .reshape(b, c, h, w))
    return qst, assign


# speedup vs baseline: 4.9335x; 1.4783x over previous
import jax
import jax.numpy as jnp
from jax import lax
from jax.experimental import pallas as pl
from jax.experimental.pallas import tpu as pltpu
from jax.experimental.pallas import tpu_sc as plsc

_INV_T = 2.0
_K = 64
_NC, _NS, _L = 2, 16, 16
_NW = _NC * _NS
_C = 384
_CHI = _C // 128
_PIX_W = _K * _CHI * 128
_GRP_W = 8 * _CHI * 128


def _sq_body(x_hbm, tab_hbm, assign_hbm, qst_hbm,
             tab_v, pp_v, qq_v, rr_v, ss_v,
             xg_v, qg_v, out0_v, out1_v, sem0, sem1):
    npix = qst_hbm.shape[0] // _C
    grp_per_w = npix // (8 * _NW)
    wid = lax.axis_index("s") * _NC + lax.axis_index("c")

    pltpu.sync_copy(tab_hbm, tab_v)
    pltpu.sync_copy(tab_hbm.at[pl.ds(0, _K)], pp_v)
    pltpu.sync_copy(tab_hbm.at[pl.ds(_K, _K)], qq_v)
    pltpu.sync_copy(tab_hbm.at[pl.ds(2 * _K, _K)], rr_v)
    pltpu.sync_copy(tab_hbm.at[pl.ds(3 * _K, _K)], ss_v)

    scal = tab_v[pl.ds(4 * _K, _L)]
    c0 = scal[0]
    inv_d = scal[1]
    cmax = scal[2]
    a0s = jnp.full((_L,), scal[3], jnp.float32)
    b0s = jnp.full((_L,), scal[4], jnp.float32)
    gs = jnp.full((_L,), scal[5], jnp.float32)
    gis = jnp.full((_L,), scal[6], jnp.float32)
    g16s = jnp.full((_L,), scal[7], jnp.float32)
    gi16s = jnp.full((_L,), scal[8], jnp.float32)

    out_bufs = (out0_v, out1_v)
    sems = (sem0, sem1)

    def group_body(g, carry):
        grp = wid * grp_per_w + g
        pltpu.sync_copy(x_hbm.at[pl.ds(grp * _GRP_W, _GRP_W)], xg_v)

        def pair_body(pair, c2):
            for b in range(2):
                wl = pair * 2 + b
                out_v = out_bufs[b]
                sem = sems[b]

                @pl.when(jnp.logical_or(g > 0, pair > 0))
                def _():
                    pltpu.make_async_copy(
                        assign_hbm.at[pl.ds(0, _PIX_W)], out_v, sem).wait()

                for chi in range(_CHI):
                    def cv_body(j, c3, chi=chi, wl=wl, out_v=out_v):
                        xoff = chi * 1024 + wl * 128 + j * _L
                        xv = xg_v[pl.ds(xoff, _L)]
                        xc = jnp.minimum(jnp.maximum(xv, c0), cmax)
                        ii = jnp.minimum(
                            ((xc - c0) * inv_d).astype(jnp.int32), _K - 1)
                        u = jnp.exp(-_INV_T * xc)
                        v = jnp.exp(_INV_T * xc)
                        pg = plsc.load_gather(pp_v, [ii])
                        qg = plsc.load_gather(qq_v, [ii])
                        rg = plsc.load_gather(rr_v, [ii])
                        sg = plsc.load_gather(ss_v, [ii])
                        inv = 1.0 / (u * pg + v * qg)
                        quant = (u * rg + v * sg) * inv
                        qg_v[pl.ds(xoff, _L)] = xv + (quant - xv)
                        uas = [(u * inv) * a0s]
                        vbs = [(v * inv) * b0s]
                        for m in range(3):
                            uas.append(uas[m] * g16s)
                            vbs.append(vbs[m] * gi16s)
                        obase = chi * 1024 + j * _L
                        for t in range(16):
                            for m in range(4):
                                k = m * 16 + t
                                e = jnp.minimum(uas[m], vbs[m])
                                out_v[pl.ds(obase + (k // 8) * 3072
                                            + (k % 8) * 128, _L)] = e
                                uas[m] = uas[m] * gs
                                vbs[m] = vbs[m] * gis
                        return c3

                    lax.fori_loop(0, 128 // _L, cv_body, 0, unroll=2)

                pltpu.async_copy(
                    out_v,
                    assign_hbm.at[pl.ds((grp * 8 + wl) * _PIX_W, _PIX_W)],
                    sem)
            return c2

        lax.fori_loop(0, 4, pair_body, 0)
        pltpu.sync_copy(qg_v, qst_hbm.at[pl.ds(grp * _GRP_W, _GRP_W)])
        return carry

    lax.fori_loop(0, grp_per_w, group_body, 0)

    pltpu.make_async_copy(assign_hbm.at[pl.ds(0, _PIX_W)], out0_v, sem0).wait()
    pltpu.make_async_copy(assign_hbm.at[pl.ds(0, _PIX_W)], out1_v, sem1).wait()


def kernel(x, centers):
    b, c, h, w = x.shape
    assert c == _C and (b * h * w) % (8 * _NW) == 0, x.shape
    n = x.size

    cf = centers.astype(jnp.float32)
    a_t = jnp.exp(_INV_T * cf)
    b_t = jnp.exp(-_INV_T * cf)
    p_t = jnp.cumsum(a_t)
    q_t = jnp.cumsum(b_t[::-1])[::-1] - b_t
    r_t = jnp.cumsum(a_t * cf)
    s_t = jnp.cumsum((b_t * cf)[::-1])[::-1] - b_t * cf
    inv_d = (_K - 1) / (cf[-1] - cf[0])
    step = (cf[-1] - cf[0]) / (_K - 1)
    scal = jnp.zeros((_K,), jnp.float32)
    scal = (scal.at[0].set(cf[0]).at[1].set(inv_d).at[2].set(cf[-1])
            .at[3].set(a_t[0]).at[4].set(b_t[0])
            .at[5].set(jnp.exp(_INV_T * step)).at[6].set(jnp.exp(-_INV_T * step))
            .at[7].set(jnp.exp(_INV_T * step * 16))
            .at[8].set(jnp.exp(-_INV_T * step * 16)))
    tab = jnp.concatenate([p_t, q_t, r_t, s_t, scal])

    x1d = (x.transpose(0, 2, 3, 1)
            .reshape(b, h, w // 8, 8, _CHI, 128)
            .transpose(0, 1, 2, 4, 3, 5)
            .reshape(-1))

    run = pl.kernel(
        _sq_body,
        out_type=[
            jax.ShapeDtypeStruct((n * _K,), jnp.float32),
            jax.ShapeDtypeStruct((n,), jnp.float32),
        ],
        mesh=plsc.VectorSubcoreMesh(core_axis_name="c", subcore_axis_name="s"),
        compiler_params=pltpu.CompilerParams(needs_layout_passes=False),
        scratch_types=[
            pltpu.VMEM((5 * _K,), jnp.float32),
            pltpu.VMEM((_K,), jnp.float32),
            pltpu.VMEM((_K,), jnp.float32),
            pltpu.VMEM((_K,), jnp.float32),
            pltpu.VMEM((_K,), jnp.float32),
            pltpu.VMEM((_GRP_W,), jnp.float32),
            pltpu.VMEM((_GRP_W,), jnp.float32),
            pltpu.VMEM((_PIX_W,), jnp.float32),
            pltpu.VMEM((_PIX_W,), jnp.float32),
            pltpu.SemaphoreType.DMA,
            pltpu.SemaphoreType.DMA,
        ],
    )
    assign1d, q1d = run(x1d, tab)

    assign = (assign1d.reshape(b, h, w, 8, _CHI, 8, 128)
              .transpose(0, 4, 6, 1, 2, 3, 5)
              .reshape(b, c, h, w, _K))
    qst = (q1d.reshape(b, h, w // 8, _CHI, 8, 128)
           .transpose(0, 3, 5, 1, 2, 4)
           .reshape(b, c, h, w))
    return qst, assign


# speedup vs baseline: 5.0296x; 1.0195x over previous
import jax
import jax.numpy as jnp
from jax import lax
from jax.experimental import pallas as pl
from jax.experimental.pallas import tpu as pltpu
from jax.experimental.pallas import tpu_sc as plsc

_INV_T = 2.0
_K = 64
_NC, _NS, _L = 2, 16, 16
_NW = _NC * _NS
_C = 384
_CHI = _C // 128
_PIX_W = _K * _CHI * 128
_GRP_W = 8 * _CHI * 128


def _sq_body(x_hbm, tab_hbm, assign_hbm, qst_hbm,
             tab_v, pp_v, qq_v, rr_v, ss_v,
             xg_v, qg_v, out0_v, out1_v, sem0, sem1):
    npix = qst_hbm.shape[0] // _C
    grp_per_w = npix // (8 * _NW)
    wid = lax.axis_index("s") * _NC + lax.axis_index("c")

    pltpu.sync_copy(tab_hbm, tab_v)
    pltpu.sync_copy(tab_hbm.at[pl.ds(0, _K)], pp_v)
    pltpu.sync_copy(tab_hbm.at[pl.ds(_K, _K)], qq_v)
    pltpu.sync_copy(tab_hbm.at[pl.ds(2 * _K, _K)], rr_v)
    pltpu.sync_copy(tab_hbm.at[pl.ds(3 * _K, _K)], ss_v)

    scal = tab_v[pl.ds(4 * _K, _L)]
    c0 = scal[0]
    inv_d = scal[1]
    cmax = scal[2]
    a0s = jnp.full((_L,), scal[3], jnp.float32)
    b0s = jnp.full((_L,), scal[4], jnp.float32)
    gs = jnp.full((_L,), scal[5], jnp.float32)
    gis = jnp.full((_L,), scal[6], jnp.float32)
    g16s = jnp.full((_L,), scal[7], jnp.float32)
    gi16s = jnp.full((_L,), scal[8], jnp.float32)

    out_bufs = (out0_v, out1_v)
    sems = (sem0, sem1)

    def group_body(g, carry):
        grp = wid * grp_per_w + g
        pltpu.sync_copy(x_hbm.at[pl.ds(grp * _GRP_W, _GRP_W)], xg_v)

        def pair_body(pair, c2):
            for b in range(2):
                wl = pair * 2 + b
                out_v = out_bufs[b]
                sem = sems[b]

                @pl.when(jnp.logical_or(g > 0, pair > 0))
                def _():
                    pltpu.make_async_copy(
                        assign_hbm.at[pl.ds(0, _PIX_W)], out_v, sem).wait()

                for chi in range(_CHI):
                    def cv_body(j, c3, chi=chi, wl=wl, out_v=out_v):
                        xoff = chi * 1024 + wl * 128 + j * _L
                        xv = xg_v[pl.ds(xoff, _L)]
                        xc = jnp.minimum(jnp.maximum(xv, c0), cmax)
                        ii = jnp.minimum(
                            ((xc - c0) * inv_d).astype(jnp.int32), _K - 1)
                        u = jnp.exp(-_INV_T * xc)
                        v = jnp.exp(_INV_T * xc)
                        pg = plsc.load_gather(pp_v, [ii])
                        qg = plsc.load_gather(qq_v, [ii])
                        rg = plsc.load_gather(rr_v, [ii])
                        sg = plsc.load_gather(ss_v, [ii])
                        inv = 1.0 / (u * pg + v * qg)
                        quant = (u * rg + v * sg) * inv
                        qg_v[pl.ds(xoff, _L)] = xv + (quant - xv)
                        uas = [(u * inv) * a0s]
                        vbs = [(v * inv) * b0s]
                        for m in range(3):
                            uas.append(uas[m] * g16s)
                            vbs.append(vbs[m] * gi16s)
                        obase = chi * 1024 + j * _L
                        for t in range(16):
                            for m in range(4):
                                k = m * 16 + t
                                e = jnp.minimum(uas[m], vbs[m])
                                out_v[pl.ds(obase + (k // 8) * 3072
                                            + (k % 8) * 128, _L)] = e
                                uas[m] = uas[m] * gs
                                vbs[m] = vbs[m] * gis
                        return c3

                    lax.fori_loop(0, 128 // _L, cv_body, 0, unroll=2)

                pltpu.async_copy(
                    out_v,
                    assign_hbm.at[pl.ds((grp * 8 + wl) * _PIX_W, _PIX_W)],
                    sem)
            return c2

        lax.fori_loop(0, 4, pair_body, 0)
        pltpu.sync_copy(qg_v, qst_hbm.at[pl.ds(grp * _GRP_W, _GRP_W)])
        return carry

    lax.fori_loop(0, grp_per_w, group_body, 0)

    pltpu.make_async_copy(assign_hbm.at[pl.ds(0, _PIX_W)], out0_v, sem0).wait()
    pltpu.make_async_copy(assign_hbm.at[pl.ds(0, _PIX_W)], out1_v, sem1).wait()


def kernel(x, centers):
    b, c, h, w = x.shape
    assert c == _C and (b * h * w) % (8 * _NW) == 0, x.shape
    n = x.size

    cf = centers.astype(jnp.float32)
    a_t = jnp.exp(_INV_T * cf)
    b_t = jnp.exp(-_INV_T * cf)
    m4 = jnp.stack([a_t, a_t * cf, b_t, b_t * cf])
    kk = jnp.arange(_K)
    lower = (kk[:, None] <= kk[None, :]).astype(jnp.float32)
    upper = (kk[:, None] > kk[None, :]).astype(jnp.float32)
    w2 = jnp.concatenate([lower, upper], axis=1)
    pr_qs = m4 @ w2
    p_t, r_t = pr_qs[0, :_K], pr_qs[1, :_K]
    q_t, s_t = pr_qs[2, _K:], pr_qs[3, _K:]
    inv_d = (_K - 1) / (cf[-1] - cf[0])
    step = (cf[-1] - cf[0]) / (_K - 1)
    g1 = jnp.exp(_INV_T * step)
    scal = jnp.stack([cf[0], inv_d, cf[-1], a_t[0], b_t[0],
                      g1, 1.0 / g1, g1 ** 16, 1.0 / g1 ** 16])
    tab = jnp.concatenate([p_t, q_t, r_t, s_t, scal,
                           jnp.zeros((_K - 9,), jnp.float32)])

    x1d = (x.transpose(0, 2, 3, 1)
            .reshape(b, h, w // 8, 8, _CHI, 128)
            .transpose(0, 1, 2, 4, 3, 5)
            .reshape(-1))

    run = pl.kernel(
        _sq_body,
        out_type=[
            jax.ShapeDtypeStruct((n * _K,), jnp.float32),
            jax.ShapeDtypeStruct((n,), jnp.float32),
        ],
        mesh=plsc.VectorSubcoreMesh(core_axis_name="c", subcore_axis_name="s"),
        compiler_params=pltpu.CompilerParams(needs_layout_passes=False),
        scratch_types=[
            pltpu.VMEM((5 * _K,), jnp.float32),
            pltpu.VMEM((_K,), jnp.float32),
            pltpu.VMEM((_K,), jnp.float32),
            pltpu.VMEM((_K,), jnp.float32),
            pltpu.VMEM((_K,), jnp.float32),
            pltpu.VMEM((_GRP_W,), jnp.float32),
            pltpu.VMEM((_GRP_W,), jnp.float32),
            pltpu.VMEM((_PIX_W,), jnp.float32),
            pltpu.VMEM((_PIX_W,), jnp.float32),
            pltpu.SemaphoreType.DMA,
            pltpu.SemaphoreType.DMA,
        ],
    )
    assign1d, q1d = run(x1d, tab)

    assign = (assign1d.reshape(b, h, w, 8, _CHI, 8, 128)
              .transpose(0, 4, 6, 1, 2, 3, 5)
              .reshape(b, c, h, w, _K))
    qst = (q1d.reshape(b, h, w // 8, _CHI, 8, 128)
           .transpose(0, 3, 5, 1, 2, 4)
           .reshape(b, c, h, w))
    return qst, assign
